# baseline (speedup 1.0000x reference)
import functools

import jax
import jax.numpy as jnp
from jax import lax
from jax.experimental import pallas as pl
from jax.experimental.pallas import tpu as pltpu
from jax.experimental.pallas import tpu_sc as plsc

_HALF = 524288


def _build_pack_kernel(n_rows, D, Dp, BI):
  H = D // 2
  grid = _HALF // BI
  hblk = _HALF // BI
  last = (n_rows - 1) // BI
  Din = ((D + 7) // 8) * 8

  def _pack_words(xt):
    a = lax.bitcast_convert_type(xt[:, :H], jnp.int32)
    b = lax.bitcast_convert_type(xt[:, H:2 * H], jnp.int32)
    a = a + 0x8000 + ((a >> 16) & 1)
    b = b + 0x8000 + ((b >> 16) & 1)
    w = (a & jnp.int32(-65536)) | lax.shift_right_logical(b, 16)
    return lax.bitcast_convert_type(w, jnp.float32)

  def pack_body(vta_ref, vtb_ref, ba_ref, bb_ref, out_ref):
    xta = jnp.transpose(vta_ref[...], (1, 0))
    xtb = jnp.transpose(vtb_ref[...], (1, 0))
    out_ref[:, :H] = _pack_words(xta)
    out_ref[:, H:H + 1] = ba_ref[...].reshape(BI, 1)
    out_ref[:, 64:64 + H] = _pack_words(xtb)
    out_ref[:, 64 + H:64 + H + 1] = bb_ref[...].reshape(BI, 1)

  return pl.pallas_call(
      pack_body,
      grid=(grid,),
      in_specs=[
          pl.BlockSpec((Din, BI), lambda g: (0, g)),
          pl.BlockSpec((Din, BI),
                       lambda g: (0, jnp.minimum(g + hblk, last))),
          pl.BlockSpec((BI,), lambda g: (g,)),
          pl.BlockSpec((BI,), lambda g: (jnp.minimum(g + hblk, last),)),
      ],
      out_specs=pl.BlockSpec((BI, Dp), lambda g: (g, 0)),
      out_shape=jax.ShapeDtypeStruct((_HALF, Dp), jnp.float32),
      compiler_params=pltpu.CompilerParams(
          dimension_semantics=("parallel",)),
  )


def _build_fm_kernel(B, n_rows, Dp, D):
  info = plsc.get_sparse_core_info()
  NC, NS, L = info.num_cores, info.num_subcores, info.num_lanes
  NW = NC * NS
  bpw = B // NW
  CH = 128
  nch = bpw // CH
  nblk = CH // L
  H = D // 2

  mesh = plsc.VectorSubcoreMesh(core_axis_name="c", subcore_axis_name="s")

  @functools.partial(
      pl.kernel,
      mesh=mesh,
      out_type=jax.ShapeDtypeStruct((B,), jnp.float32),
      compiler_params=pltpu.CompilerParams(
          needs_layout_passes=False, use_tc_tiling_on_sc=False),
      scratch_types=[
          pltpu.VMEM((nch, CH), jnp.int32),
          pltpu.VMEM((nch, CH), jnp.int32),
          pltpu.VMEM((bpw,), jnp.int32),
          pltpu.VMEM((bpw,), jnp.int32),
          pltpu.VMEM((3, CH, Dp), jnp.float32),
          pltpu.VMEM((3, CH, Dp), jnp.float32),
          pltpu.VMEM((bpw,), jnp.float32),
          pltpu.VMEM((L,), jnp.float32),
          pltpu.SemaphoreType.DMA,
          pltpu.SemaphoreType.DMA,
          pltpu.SemaphoreType.DMA,
      ],
  )
  def fm(i_hbm, j_hbm, hi_hbm, hj_hbm, t_hbm, c_hbm, out_hbm,
         ii_v, jj_v, hi_v, hj_v, vi_v, vj_v, o_v, c_v, sem0, sem1, sem2):
    wid = lax.axis_index("s") * NC + lax.axis_index("c")
    base = wid * bpw
    sems = (sem0, sem1, sem2)

    pltpu.sync_copy(c_hbm, c_v)
    pltpu.sync_copy(hi_hbm.at[pl.ds(base, bpw)], hi_v)
    pltpu.sync_copy(hj_hbm.at[pl.ds(base, bpw)], hj_v)
    for k in range(nch):
      pltpu.sync_copy(i_hbm.at[pl.ds(base + k * CH, CH)], ii_v.at[k])
      pltpu.sync_copy(j_hbm.at[pl.ds(base + k * CH, CH)], jj_v.at[k])

    def start(k):
      p = k % 3
      return (
          pltpu.async_copy(t_hbm.at[ii_v.at[k]], vi_v.at[p], sems[p]),
          pltpu.async_copy(t_hbm.at[jj_v.at[k]], vj_v.at[p], sems[p]),
      )

    iota = lax.iota(jnp.int32, L)
    cv = c_v[...]
    himask = jnp.full((L,), -65536, dtype=jnp.int32)

    inflight = {0: start(0), 1: start(1)}
    for k in range(nch):
      for cp in inflight.pop(k):
        cp.wait()
      if k + 2 < nch:
        inflight[k + 2] = start(k + 2)
      p = k % 3
      pvec = jnp.full((L,), p, dtype=jnp.int32)

      def blk_body(b, carry):
        rows = iota + b * L
        osl = pl.ds(k * CH + b * L, L)
        hbi = hi_v[osl]
        hbj = hj_v[osl]

        def d_body(w, acc):
          wi = plsc.load_gather(vi_v, [pvec, rows, hbi + w])
          wj = plsc.load_gather(vj_v, [pvec, rows, hbj + w])
          bi_ = plsc.bitcast(wi, jnp.int32)
          bj_ = plsc.bitcast(wj, jnp.int32)
          hi = plsc.bitcast(bi_ & himask, jnp.float32)
          hj = plsc.bitcast(bj_ & himask, jnp.float32)
          lo_i = plsc.bitcast(bi_ << 16, jnp.float32)
          lo_j = plsc.bitcast(bj_ << 16, jnp.float32)
          return acc + hi * hj + lo_i * lo_j

        acc = lax.fori_loop(0, H, d_body, jnp.zeros((L,), jnp.float32),
                            unroll=2)
        bi = plsc.load_gather(vi_v, [pvec, rows, hbi + H])
        bj = plsc.load_gather(vj_v, [pvec, rows, hbj + H])
        o_v[osl] = cv + bi + bj + acc
        return carry

      lax.fori_loop(0, nblk, blk_body, 0)

    pltpu.sync_copy(o_v, out_hbm.at[pl.ds(base, bpw)])

  return fm


def kernel(i, j, y, V, bias, c):
  del y
  B = i.shape[0]
  n_rows, D = V.shape
  Dp = 128
  BI = 16384
  pack = _build_pack_kernel(n_rows, D, Dp, BI)
  vt = jnp.transpose(V)
  b1 = bias.reshape(n_rows)
  table = pack(vt, vt, b1, b1)
  fm = _build_fm_kernel(B, n_rows, Dp, D)
  c16 = jnp.broadcast_to(c.astype(jnp.float32), (16,))
  i32 = i.astype(jnp.int32)
  j32 = j.astype(jnp.int32)
  ge_i = i32 >= _HALF
  ge_j = j32 >= _HALF
  ie = jnp.where(ge_i, i32 - _HALF, i32)
  je = jnp.where(ge_j, j32 - _HALF, j32)
  hi = jnp.where(ge_i, 64, 0).astype(jnp.int32)
  hj = jnp.where(ge_j, 64, 0).astype(jnp.int32)
  return fm(ie, je, hi, hj, table, c16)


# speedup vs baseline: 1.3955x; 1.3955x over previous
import functools

import jax
import jax.numpy as jnp
from jax import lax
from jax.experimental import pallas as pl
from jax.experimental.pallas import tpu as pltpu
from jax.experimental.pallas import tpu_sc as plsc


def _build_pack_kernel(n_rows, D, Dp, BI):
  grid = (n_rows + BI - 1) // BI
  Din = ((D + 7) // 8) * 8

  def pack_body(vt_ref, b_ref, out_ref):
    x = vt_ref[...]
    xt = jnp.transpose(x, (1, 0))
    out_ref[:, :D] = xt[:, :D]
    out_ref[:, D:D + 1] = b_ref[...].reshape(BI, 1)

  return pl.pallas_call(
      pack_body,
      grid=(grid,),
      in_specs=[
          pl.BlockSpec((Din, BI), lambda g: (0, g)),
          pl.BlockSpec((BI,), lambda g: (g,)),
      ],
      out_specs=pl.BlockSpec((BI, Dp), lambda g: (g, 0)),
      out_shape=jax.ShapeDtypeStruct((n_rows, Dp), jnp.float32),
      compiler_params=pltpu.CompilerParams(
          dimension_semantics=("parallel",)),
  )


def _build_fm_kernel(B, n_rows, Dp, D):
  info = plsc.get_sparse_core_info()
  NC, NS, L = info.num_cores, info.num_subcores, info.num_lanes
  NW = NC * NS
  bpw = B // NW
  CH = 128
  nch = bpw // CH
  nblk = CH // L

  mesh = plsc.VectorSubcoreMesh(core_axis_name="c", subcore_axis_name="s")

  @functools.partial(
      pl.kernel,
      mesh=mesh,
      out_type=jax.ShapeDtypeStruct((B,), jnp.float32),
      compiler_params=pltpu.CompilerParams(
          needs_layout_passes=False, use_tc_tiling_on_sc=False),
      scratch_types=[
          pltpu.VMEM((nch, CH), jnp.int32),
          pltpu.VMEM((nch, CH), jnp.int32),
          pltpu.VMEM((3, CH, Dp), jnp.float32),
          pltpu.VMEM((3, CH, Dp), jnp.float32),
          pltpu.VMEM((bpw,), jnp.float32),
          pltpu.VMEM((L,), jnp.float32),
          pltpu.SemaphoreType.DMA,
          pltpu.SemaphoreType.DMA,
          pltpu.SemaphoreType.DMA,
      ],
  )
  def fm(i_hbm, j_hbm, t_hbm, c_hbm, out_hbm,
         ii_v, jj_v, vi_v, vj_v, o_v, c_v, sem0, sem1, sem2):
    wid = lax.axis_index("s") * NC + lax.axis_index("c")
    base = wid * bpw
    sems = (sem0, sem1, sem2)

    pltpu.sync_copy(c_hbm, c_v)
    for k in range(nch):
      pltpu.sync_copy(i_hbm.at[pl.ds(base + k * CH, CH)], ii_v.at[k])
      pltpu.sync_copy(j_hbm.at[pl.ds(base + k * CH, CH)], jj_v.at[k])

    def start(k):
      p = k % 3
      return (
          pltpu.async_copy(t_hbm.at[ii_v.at[k]], vi_v.at[p], sems[p]),
          pltpu.async_copy(t_hbm.at[jj_v.at[k]], vj_v.at[p], sems[p]),
      )

    iota = lax.iota(jnp.int32, L)
    cv = c_v[...]
    bcol = jnp.full((L,), D, dtype=jnp.int32)

    inflight = {0: start(0), 1: start(1)}
    for k in range(nch):
      for cp in inflight.pop(k):
        cp.wait()
      if k + 2 < nch:
        inflight[k + 2] = start(k + 2)
      p = k % 3
      pvec = jnp.full((L,), p, dtype=jnp.int32)

      def blk_body(b, carry):
        rows = iota + b * L

        def d_body(d, acc):
          cols = jnp.full((L,), d, dtype=jnp.int32)
          a = plsc.load_gather(vi_v, [pvec, rows, cols])
          bb = plsc.load_gather(vj_v, [pvec, rows, cols])
          return acc + a * bb

        acc = lax.fori_loop(0, D, d_body, jnp.zeros((L,), jnp.float32),
                            unroll=4)
        bi = plsc.load_gather(vi_v, [pvec, rows, bcol])
        bj = plsc.load_gather(vj_v, [pvec, rows, bcol])
        o_v[pl.ds(k * CH + b * L, L)] = cv + bi + bj + acc
        return carry

      lax.fori_loop(0, nblk, blk_body, 0)

    pltpu.sync_copy(o_v, out_hbm.at[pl.ds(base, bpw)])

  return fm


def kernel(i, j, y, V, bias, c):
  del y
  B = i.shape[0]
  n_rows, D = V.shape
  Dp = 128
  BI = 24576
  pack = _build_pack_kernel(n_rows, D, Dp, BI)
  table = pack(jnp.transpose(V), bias.reshape(n_rows))
  fm = _build_fm_kernel(B, n_rows, Dp, D)
  c16 = jnp.broadcast_to(c.astype(jnp.float32), (16,))
  return fm(i.astype(jnp.int32), j.astype(jnp.int32), table, c16)
